# baseline (speedup 1.0000x reference)
import jax
import jax.numpy as jnp
from jax import lax
from jax.experimental import pallas as pl
from jax.experimental.pallas import tpu as pltpu
from jax.experimental.pallas import tpu_sc as plsc

B = 4096
SEQ = 200
D = 64
V = 1000000
N = B * SEQ
NW = 32
BLK = B // NW
DEPTH = 4
K = DEPTH - 1
TDEPTH = 2


def _emb_body(ids_hbm, table_hbm, out_hbm, idx_v, rows_v, trans_v, gsem, osem):
    wid = lax.axis_index("s") * 2 + lax.axis_index("c")
    base = wid * BLK * SEQ
    pltpu.sync_copy(ids_hbm.at[pl.ds(base, BLK * SEQ)], idx_v)

    lanes = lax.iota(jnp.int32, 16)

    def gather(s, b):
        def grp(g, _):
            offs = (lanes + g * 16) * SEQ + s
            jv = plsc.load_gather(idx_v, [offs])
            for t16 in range(16):
                pltpu.async_copy(
                    table_hbm.at[jv[t16]],
                    rows_v.at[b, g * 16 + t16, pl.ds(0, D)],
                    gsem.at[b],
                )
            return 0

        lax.fori_loop(0, BLK // 16, grp, 0, unroll=False)

    def wait_gather(b):
        pltpu.make_async_copy(
            out_hbm.at[0, :, pl.ds(0, BLK)], trans_v.at[0], gsem.at[b]
        ).wait()

    def transpose(b, t):
        def dloop(d, _):
            dv = jnp.full((16,), 0, jnp.int32) + d
            for g in range(BLK // 16):
                x = plsc.load_gather(rows_v.at[b], [lanes + g * 16, dv])
                trans_v[t, d, pl.ds(g * 16, 16)] = x
            return 0

        lax.fori_loop(0, D, dloop, 0, unroll=False)

    def put(s, t):
        pltpu.async_copy(
            trans_v.at[t],
            out_hbm.at[s, :, pl.ds(wid * BLK, BLK)],
            osem.at[t],
        )

    def wait_put(t):
        pltpu.make_async_copy(
            trans_v.at[t],
            out_hbm.at[0, :, pl.ds(wid * BLK, BLK)],
            osem.at[t],
        ).wait()

    for j in range(K):
        gather(j, j)

    def body(g, _):
        for db in range(DEPTH):
            s = g * DEPTH + db
            b = db
            t = db % TDEPTH
            @pl.when(s + K < SEQ)
            def _():
                gather(s + K, (db + K) % DEPTH)

            wait_gather(b)

            @pl.when(s >= TDEPTH)
            def _():
                wait_put(t)

            transpose(b, t)
            put(s, t)
        return 0

    lax.fori_loop(0, SEQ // DEPTH, body, 0, unroll=False)

    for t in range(TDEPTH):
        wait_put(t)


@jax.jit
def kernel(token_ids, embed_weight):
    ids_flat = token_ids.reshape(-1)
    mesh = plsc.VectorSubcoreMesh(core_axis_name="c", subcore_axis_name="s")
    out = pl.kernel(
        _emb_body,
        out_type=jax.ShapeDtypeStruct((SEQ, D, B), jnp.float32),
        mesh=mesh,
        scratch_types=[
            pltpu.VMEM((BLK * SEQ,), jnp.int32),
            pltpu.VMEM((DEPTH, BLK, 2 * D), jnp.float32),
            pltpu.VMEM((TDEPTH, D, BLK), jnp.float32),
            pltpu.SemaphoreType.DMA((DEPTH,)),
            pltpu.SemaphoreType.DMA((TDEPTH,)),
        ],
        compiler_params=pltpu.CompilerParams(needs_layout_passes=False),
    )(ids_flat, embed_weight)
    return jnp.transpose(out, (2, 0, 1))


# speedup vs baseline: 2.0066x; 2.0066x over previous
import jax
import jax.numpy as jnp
from jax import lax
from jax.experimental import pallas as pl
from jax.experimental.pallas import tpu as pltpu
from jax.experimental.pallas import tpu_sc as plsc

B = 4096
SEQ = 200
D = 64
V = 1000000
N = B * SEQ
NW = 32
PER_W = N // NW
CHUNK = 128
NCHUNK = PER_W // CHUNK
DEPTH = 5
K = DEPTH - 1


def _emb_body(ids_hbm, table_hbm, out_hbm, idx_v, rows_v, gsem, osem):
    wid = lax.axis_index("s") * 2 + lax.axis_index("c")
    base = wid * PER_W
    pltpu.sync_copy(ids_hbm.at[pl.ds(base, PER_W)], idx_v)

    def gather(i, b):
        def grp(g, _):
            jv = idx_v[pl.ds(i * CHUNK + g * 16, 16)]
            for t16 in range(16):
                pltpu.async_copy(
                    table_hbm.at[jv[t16]],
                    rows_v.at[b, g * 16 + t16],
                    gsem.at[b],
                )
            return 0

        lax.fori_loop(0, CHUNK // 16, grp, 0, unroll=False)

    def wait_gather(b):
        pltpu.make_async_copy(
            table_hbm.at[pl.ds(0, CHUNK)], rows_v.at[b], gsem.at[b]
        ).wait()

    def put(i, b):
        pltpu.async_copy(
            rows_v.at[b],
            out_hbm.at[pl.ds(base + i * CHUNK, CHUNK)],
            osem.at[b],
        )

    def wait_put(b):
        pltpu.make_async_copy(
            rows_v.at[b],
            out_hbm.at[pl.ds(base, CHUNK)],
            osem.at[b],
        ).wait()

    for j in range(K):
        gather(j, j)

    def body(g, _):
        for db in range(DEPTH):
            i = g * DEPTH + db
            b = db
            bn = (db + K) % DEPTH
            @pl.when(i + K < NCHUNK)
            def _():
                @pl.when(i >= 1)
                def _():
                    wait_put(bn)

                gather(i + K, bn)

            wait_gather(b)
            put(i, b)
        return 0

    lax.fori_loop(0, NCHUNK // DEPTH, body, 0, unroll=False)

    for c in range(NCHUNK - DEPTH, NCHUNK):
        wait_put(c % DEPTH)


@jax.jit
def kernel(token_ids, embed_weight):
    ids_flat = token_ids.reshape(-1)
    mesh = plsc.VectorSubcoreMesh(core_axis_name="c", subcore_axis_name="s")
    out = pl.kernel(
        _emb_body,
        out_type=jax.ShapeDtypeStruct((N, D), jnp.float32),
        mesh=mesh,
        scratch_types=[
            pltpu.VMEM((PER_W,), jnp.int32),
            pltpu.VMEM((DEPTH, CHUNK, D), jnp.float32),
            pltpu.SemaphoreType.DMA((DEPTH,)),
            pltpu.SemaphoreType.DMA((DEPTH,)),
        ],
    )(ids_flat, embed_weight)
    return out.reshape(B, SEQ, D)


# speedup vs baseline: 2.3738x; 1.1830x over previous
import jax
import jax.numpy as jnp
from jax import lax
from jax.experimental import pallas as pl
from jax.experimental.pallas import tpu as pltpu
from jax.experimental.pallas import tpu_sc as plsc

B = 4096
SEQ = 200
D = 64
V = 1000000
N = B * SEQ
NW = 32
PER_W = N // NW
CHUNK = 128
NCHUNK = PER_W // CHUNK
DEPTH = 5
K = DEPTH - 1


def _emb_body(ids_hbm, table_hbm, out_hbm, idx_v, rows_v, gsem, osem):
    wid = lax.axis_index("s") * 2 + lax.axis_index("c")
    base = wid * PER_W
    pltpu.sync_copy(ids_hbm.at[pl.ds(base, PER_W)], idx_v)

    def gather(i, b):
        def grp(g, _):
            jv = idx_v[pl.ds(i * CHUNK + g * 16, 16)]
            jhi = lax.shift_right_logical(jv, 3)
            jlo = lax.rem(jv, 8)
            for t16 in range(16):
                pltpu.async_copy(
                    table_hbm.at[jhi[t16], jlo[t16]],
                    rows_v.at[b, g * 16 + t16],
                    gsem.at[b],
                )
            return 0

        lax.fori_loop(0, CHUNK // 16, grp, 0, unroll=False)

    def wait_gather(b):
        pltpu.make_async_copy(
            out_hbm.at[pl.ds(0, CHUNK)], rows_v.at[b], gsem.at[b]
        ).wait()

    def put(i, b):
        pltpu.async_copy(
            rows_v.at[b],
            out_hbm.at[pl.ds(base + i * CHUNK, CHUNK)],
            osem.at[b],
        )

    def wait_put(b):
        pltpu.make_async_copy(
            rows_v.at[b],
            out_hbm.at[pl.ds(base, CHUNK)],
            osem.at[b],
        ).wait()

    for j in range(K):
        gather(j, j)

    def body(g, _):
        for db in range(DEPTH):
            i = g * DEPTH + db
            b = db
            bn = (db + K) % DEPTH
            @pl.when(i + K < NCHUNK)
            def _():
                @pl.when(i >= 1)
                def _():
                    wait_put(bn)

                gather(i + K, bn)

            wait_gather(b)
            put(i, b)
        return 0

    lax.fori_loop(0, NCHUNK // DEPTH, body, 0, unroll=False)

    for c in range(NCHUNK - DEPTH, NCHUNK):
        wait_put(c % DEPTH)


@jax.jit
def kernel(token_ids, embed_weight):
    ids_flat = token_ids.reshape(-1)
    mesh = plsc.VectorSubcoreMesh(core_axis_name="c", subcore_axis_name="s")
    out = pl.kernel(
        _emb_body,
        out_type=jax.ShapeDtypeStruct((N, D), jnp.float32),
        mesh=mesh,
        scratch_types=[
            pltpu.VMEM((PER_W,), jnp.int32),
            pltpu.VMEM((DEPTH, CHUNK, D), jnp.float32),
            pltpu.SemaphoreType.DMA((DEPTH,)),
            pltpu.SemaphoreType.DMA((DEPTH,)),
        ],
    )(ids_flat, embed_weight.reshape(V // 8, 8, D))
    return out.reshape(B, SEQ, D)
